# Initial kernel scaffold; baseline (speedup 1.0000x reference)
import math

import jax
import jax.numpy as jnp
from jax import lax
from jax.experimental import pallas as pl
from jax.experimental.pallas import tpu as pltpu
from jax.experimental.pallas import tpu_sc as plsc

_N = 10000
_E = 160000
_H = 128
_TDIM = 64
_B = 256

_NPAD = 10240
_DUMMY = _N
_NSC = 2
_NSUB = 16
_NTILES = _NSC * _NSUB
_CHUNK = 128
_NCH = 40
_EPAD = _NTILES * _NCH * _CHUNK
_RPT = _NPAD // _NSUB

_mesh = plsc.VectorSubcoreMesh(core_axis_name="c", subcore_axis_name="s")


def _deg_body(dst_hbm, ones_hbm, zeros_hbm, out_hbm, idx_v, ones_v, acc_sh):
    cid = lax.axis_index("c")
    sid = lax.axis_index("s")
    wid = cid * _NSUB + sid
    pltpu.sync_copy(zeros_hbm, acc_sh.at[pl.ds(sid * _RPT, _RPT)])
    pltpu.sync_copy(ones_hbm, ones_v)
    pltpu.sync_copy(dst_hbm.at[wid], idx_v)
    plsc.subcore_barrier()

    def body(j, carry):
        pltpu.sync_copy(ones_v, acc_sh.at[idx_v.at[j]], add=True)
        return carry

    lax.fori_loop(0, _NCH, body, 0)
    plsc.subcore_barrier()
    pltpu.sync_copy(acc_sh.at[pl.ds(sid * _RPT, _RPT)],
                    out_hbm.at[cid, pl.ds(sid * _RPT, _RPT)])


_deg_call = pl.kernel(
    _deg_body,
    out_type=jax.ShapeDtypeStruct((_NSC, _NPAD, 16), jnp.float32),
    mesh=_mesh,
    scratch_types=[
        pltpu.VMEM((_NCH, _CHUNK), jnp.int32),
        pltpu.VMEM((_CHUNK, 16), jnp.float32),
        pltpu.VMEM_SHARED((_NPAD, 16), jnp.float32),
    ],
)


def _spmm_body(y_hbm, src_hbm, dst_hbm, zeros_hbm, out_hbm,
               sidx_v, didx_v, rows_v, acc_sh, sem):
    cid = lax.axis_index("c")
    sid = lax.axis_index("s")
    wid = cid * _NSUB + sid
    pltpu.sync_copy(zeros_hbm, acc_sh.at[pl.ds(sid * _RPT, _RPT)])
    pltpu.sync_copy(src_hbm.at[wid], sidx_v)
    pltpu.sync_copy(dst_hbm.at[wid], didx_v)
    plsc.subcore_barrier()

    def body(j, carry):
        pltpu.async_copy(y_hbm.at[sidx_v.at[j]], rows_v, sem).wait()
        pltpu.sync_copy(rows_v, acc_sh.at[didx_v.at[j]], add=True)
        return carry

    lax.fori_loop(0, _NCH, body, 0)
    plsc.subcore_barrier()
    pltpu.sync_copy(acc_sh.at[pl.ds(sid * _RPT, _RPT)],
                    out_hbm.at[cid, pl.ds(sid * _RPT, _RPT)])


_spmm_call = pl.kernel(
    _spmm_body,
    out_type=jax.ShapeDtypeStruct((_NSC, _NPAD, _H), jnp.float32),
    mesh=_mesh,
    scratch_types=[
        pltpu.VMEM((_NCH, _CHUNK), jnp.int32),
        pltpu.VMEM((_NCH, _CHUNK), jnp.int32),
        pltpu.VMEM((_CHUNK, _H), jnp.float32),
        pltpu.VMEM_SHARED((_NPAD, _H), jnp.float32),
        pltpu.SemaphoreType.DMA,
    ],
)


def _edge_body(p_hbm, q_hbm, src_hbm, dst_hbm, out_hbm,
               sidx_v, didx_v, pbuf, qbuf, sem1, sem2):
    cid = lax.axis_index("c")
    sid = lax.axis_index("s")
    wid = cid * _NSUB + sid
    pltpu.sync_copy(src_hbm.at[wid], sidx_v)
    pltpu.sync_copy(dst_hbm.at[wid], didx_v)

    def body(j, carry):
        cp1 = pltpu.async_copy(p_hbm.at[sidx_v.at[j]], pbuf, sem1)
        cp2 = pltpu.async_copy(q_hbm.at[didx_v.at[j]], qbuf, sem2)
        cp1.wait()
        cp2.wait()

        def rbody(r, c2):
            for c in range(_H // 16):
                sl = pl.ds(c * 16, 16)
                pbuf[r, sl] = pbuf[r, sl] + qbuf[r, sl]
            return c2

        lax.fori_loop(0, _CHUNK, rbody, 0)
        pltpu.sync_copy(pbuf, out_hbm.at[pl.ds((wid * _NCH + j) * _CHUNK, _CHUNK)])
        return carry

    lax.fori_loop(0, _NCH, body, 0)


_edge_call = pl.kernel(
    _edge_body,
    out_type=jax.ShapeDtypeStruct((_EPAD, _H), jnp.float32),
    mesh=_mesh,
    scratch_types=[
        pltpu.VMEM((_NCH, _CHUNK), jnp.int32),
        pltpu.VMEM((_NCH, _CHUNK), jnp.int32),
        pltpu.VMEM((_CHUNK, _H), jnp.float32),
        pltpu.VMEM((_CHUNK, _H), jnp.float32),
        pltpu.SemaphoreType.DMA,
        pltpu.SemaphoreType.DMA,
    ],
)


_RB = 512
_GRID = _NPAD // _RB


def _dis_from(degp_ref):
    deg = degp_ref[0, :, 0:1] + degp_ref[1, :, 0:1] + 1.0
    return lax.rsqrt(deg)


def _tc1_body(x_ref, w1_ref, degp_ref, y1_ref):
    dis = _dis_from(degp_ref)
    y1_ref[...] = jnp.dot(x_ref[...], w1_ref[...],
                          preferred_element_type=jnp.float32) * dis


def _tc2_body(zp_ref, y1_ref, degp_ref, w2_ref, b1_ref, y2_ref):
    dis = _dis_from(degp_ref)
    h1 = jnp.maximum(
        (zp_ref[0] + zp_ref[1] + y1_ref[...]) * dis + b1_ref[...], 0.0)
    y2_ref[...] = jnp.dot(h1, w2_ref[...],
                          preferred_element_type=jnp.float32) * dis


def _tc3_body(zp_ref, y2_ref, degp_ref, b2_ref, tg_ref, bv_ref,
              wt_ref, bt_ref, wn_ref, bn_ref, we1t_ref, we1b_ref, be1_ref,
              nl_ref, p_ref, q_ref):
    dis = _dis_from(degp_ref)
    h2 = jnp.maximum(
        (zp_ref[0] + zp_ref[1] + y2_ref[...]) * dis + b2_ref[...], 0.0)
    half = _TDIM // 2
    k = lax.broadcasted_iota(jnp.float32, (1, half), 1)
    freqs = jnp.exp(k * (-math.log(10000.0) / half))
    args = tg_ref[...] * freqs
    emb = jnp.concatenate([jnp.sin(args), jnp.cos(args)], axis=-1)
    pre = jnp.dot(emb, wt_ref[...], preferred_element_type=jnp.float32) + bt_ref[...]
    temb = pre * (1.0 / (1.0 + jnp.exp(-pre)))
    bv = bv_ref[0, 0, :]
    onehot = (bv[:, None] == lax.broadcasted_iota(jnp.int32, (_RB, _B), 1)
              ).astype(jnp.float32)
    h = h2 + jnp.dot(onehot, temb, preferred_element_type=jnp.float32)
    nl_ref[...] = jnp.dot(h, wn_ref[...], preferred_element_type=jnp.float32) + bn_ref[...]
    p_ref[...] = jnp.dot(h, we1t_ref[...], preferred_element_type=jnp.float32) + be1_ref[...]
    q_ref[...] = jnp.dot(h, we1b_ref[...], preferred_element_type=jnp.float32)


_EB = 2048


def _tc4_body(r_ref, we2_ref, be2_ref, out_ref):
    r = r_ref[...]
    s = r * (1.0 / (1.0 + jnp.exp(-r)))
    out_ref[...] = jnp.dot(s, we2_ref[...],
                           preferred_element_type=jnp.float32) + be2_ref[...]


def _row_spec():
    return pl.BlockSpec((_RB, _H), lambda i: (i, 0))


def _full(shape):
    return pl.BlockSpec(shape, lambda i: tuple(0 for _ in shape))


_degp_spec = pl.BlockSpec((_NSC, _RB, 16), lambda i: (0, i, 0))
_zp_spec = pl.BlockSpec((_NSC, _RB, _H), lambda i: (0, i, 0))


def kernel(node_x_oh, edge_index, t_graph, batch_vec,
           W1, b1, W2, b2, Wt, bt, Wn, bn, We1, be1, We2, be2):
    f32 = jnp.float32
    x = jnp.pad(node_x_oh, ((0, _NPAD - _N), (0, 0)))
    pad = jnp.full((_EPAD - _E,), _DUMMY, jnp.int32)
    src_r = jnp.concatenate([edge_index[0], pad]).reshape(_NTILES, _NCH, _CHUNK)
    dst_r = jnp.concatenate([edge_index[1], pad]).reshape(_NTILES, _NCH, _CHUNK)
    batch_r = jnp.pad(batch_vec, (0, _NPAD - _N)).reshape(_GRID, 1, _RB)
    tg = t_graph.reshape(_B, 1)
    ones16 = jnp.ones((_CHUNK, 16), f32)
    zeros16 = jnp.zeros((_RPT, 16), f32)
    zerosH = jnp.zeros((_RPT, _H), f32)
    b1r = b1.reshape(1, _H)
    b2r = b2.reshape(1, _H)
    btr = bt.reshape(1, _H)
    bnr = bn.reshape(1, _H)
    be1r = be1.reshape(1, _H)
    be2r = be2.reshape(1, 16)
    We1t = We1[:_H]
    We1b = We1[_H:]

    degp = _deg_call(dst_r, ones16, zeros16)

    y1 = pl.pallas_call(
        _tc1_body,
        grid=(_GRID,),
        in_specs=[_row_spec(), _full((_H, _H)), _degp_spec],
        out_specs=_row_spec(),
        out_shape=jax.ShapeDtypeStruct((_NPAD, _H), f32),
    )(x, W1, degp)

    z1 = _spmm_call(y1, src_r, dst_r, zerosH)

    y2 = pl.pallas_call(
        _tc2_body,
        grid=(_GRID,),
        in_specs=[_zp_spec, _row_spec(), _degp_spec, _full((_H, _H)),
                  _full((1, _H))],
        out_specs=_row_spec(),
        out_shape=jax.ShapeDtypeStruct((_NPAD, _H), f32),
    )(z1, y1, degp, W2, b1r)

    z2 = _spmm_call(y2, src_r, dst_r, zerosH)

    nl, p, q = pl.pallas_call(
        _tc3_body,
        grid=(_GRID,),
        in_specs=[_zp_spec, _row_spec(), _degp_spec, _full((1, _H)),
                  _full((_B, 1)), pl.BlockSpec((1, 1, _RB), lambda i: (i, 0, 0)),
                  _full((_TDIM, _H)), _full((1, _H)),
                  _full((_H, _H)), _full((1, _H)),
                  _full((_H, _H)), _full((_H, _H)), _full((1, _H))],
        out_specs=[_row_spec(), _row_spec(), _row_spec()],
        out_shape=[jax.ShapeDtypeStruct((_NPAD, _H), f32)] * 3,
    )(z2, y2, degp, b2r, tg, batch_r, Wt, btr, Wn, bnr, We1t, We1b, be1r)

    r = _edge_call(p, q, src_r, dst_r)

    el = pl.pallas_call(
        _tc4_body,
        grid=(_EPAD // _EB,),
        in_specs=[pl.BlockSpec((_EB, _H), lambda i: (i, 0)),
                  _full((_H, 16)), _full((1, 16))],
        out_specs=pl.BlockSpec((_EB, 16), lambda i: (i, 0)),
        out_shape=jax.ShapeDtypeStruct((_EPAD, 16), f32),
    )(r, We2, be2r)

    return (nl[:_N], el[:_E])


# speedup vs baseline: 4.2332x; 4.2332x over previous
import math

import jax
import jax.numpy as jnp
from jax import lax
from jax.experimental import pallas as pl
from jax.experimental.pallas import tpu as pltpu
from jax.experimental.pallas import tpu_sc as plsc

_N = 10000
_E = 160000
_H = 128
_TDIM = 64
_B = 256

_NPAD = 10240
_DUMMY = _N
_NSC = 2
_NSUB = 16
_NTILES = _NSC * _NSUB
_CHUNK = 128
_NCH = 40
_EPAD = _NTILES * _NCH * _CHUNK
_RPT = _NPAD // _NSUB

_mesh = plsc.VectorSubcoreMesh(core_axis_name="c", subcore_axis_name="s")


def _deg_body(dst_hbm, ones_hbm, zeros_hbm, out_hbm, dbuf, ones_v, acc_sh,
              sem_i, sem_s):
    cid = lax.axis_index("c")
    sid = lax.axis_index("s")
    wid = cid * _NSUB + sid
    pltpu.sync_copy(zeros_hbm, acc_sh.at[pl.ds(sid * _RPT, _RPT)])
    pltpu.sync_copy(ones_hbm, ones_v)
    plsc.subcore_barrier()

    def body(j, carry):
        pltpu.async_copy(dst_hbm.at[wid, j], dbuf, sem_i).wait()
        pltpu.async_copy(ones_v, acc_sh.at[dbuf], sem_s, add=True).wait()
        return carry

    lax.fori_loop(0, _NCH, body, 0)
    plsc.subcore_barrier()
    pltpu.sync_copy(acc_sh.at[pl.ds(sid * _RPT, _RPT)],
                    out_hbm.at[cid, pl.ds(sid * _RPT, _RPT)])


_deg_call = pl.kernel(
    _deg_body,
    out_type=jax.ShapeDtypeStruct((_NSC, _NPAD, _H), jnp.float32),
    mesh=_mesh,
    scratch_types=[
        pltpu.VMEM((_CHUNK,), jnp.int32),
        pltpu.VMEM((_CHUNK, _H), jnp.float32),
        pltpu.VMEM_SHARED((_NPAD, _H), jnp.float32),
        pltpu.SemaphoreType.DMA,
        pltpu.SemaphoreType.DMA,
    ],
)


def _spmm_body(y_hbm, src_hbm, dst_hbm, zeros_hbm, out_hbm,
               sbuf, dbuf, rows_v, acc_sh, sem):
    cid = lax.axis_index("c")
    sid = lax.axis_index("s")
    wid = cid * _NSUB + sid
    pltpu.sync_copy(zeros_hbm, acc_sh.at[pl.ds(sid * _RPT, _RPT)])
    plsc.subcore_barrier()

    def body(j, carry):
        pltpu.sync_copy(src_hbm.at[wid, j], sbuf)
        pltpu.sync_copy(dst_hbm.at[wid, j], dbuf)
        pltpu.async_copy(y_hbm.at[sbuf], rows_v, sem).wait()
        pltpu.sync_copy(rows_v, acc_sh.at[dbuf], add=True)
        return carry

    lax.fori_loop(0, _NCH, body, 0)
    plsc.subcore_barrier()
    pltpu.sync_copy(acc_sh.at[pl.ds(sid * _RPT, _RPT)],
                    out_hbm.at[cid, pl.ds(sid * _RPT, _RPT)])


_spmm_call = pl.kernel(
    _spmm_body,
    out_type=jax.ShapeDtypeStruct((_NSC, _NPAD, _H), jnp.float32),
    mesh=_mesh,
    scratch_types=[
        pltpu.VMEM((_CHUNK,), jnp.int32),
        pltpu.VMEM((_CHUNK,), jnp.int32),
        pltpu.VMEM((_CHUNK, _H), jnp.float32),
        pltpu.VMEM_SHARED((_NPAD, _H), jnp.float32),
        pltpu.SemaphoreType.DMA,
    ],
)


def _edge_body(p_hbm, q_hbm, src_hbm, dst_hbm, out_hbm,
               sbuf, dbuf, pbuf, qbuf, sem1, sem2):
    cid = lax.axis_index("c")
    sid = lax.axis_index("s")
    wid = cid * _NSUB + sid

    def body(j, carry):
        pltpu.sync_copy(src_hbm.at[wid, j], sbuf)
        pltpu.sync_copy(dst_hbm.at[wid, j], dbuf)
        cp1 = pltpu.async_copy(p_hbm.at[sbuf], pbuf, sem1)
        cp2 = pltpu.async_copy(q_hbm.at[dbuf], qbuf, sem2)
        cp1.wait()
        cp2.wait()

        def rbody(r, c2):
            for c in range(_H // 16):
                sl = pl.ds(c * 16, 16)
                pbuf[r, sl] = pbuf[r, sl] + qbuf[r, sl]
            return c2

        lax.fori_loop(0, _CHUNK, rbody, 0)
        pltpu.sync_copy(pbuf, out_hbm.at[pl.ds((wid * _NCH + j) * _CHUNK, _CHUNK)])
        return carry

    lax.fori_loop(0, _NCH, body, 0)


_edge_call = pl.kernel(
    _edge_body,
    out_type=jax.ShapeDtypeStruct((_EPAD, _H), jnp.float32),
    mesh=_mesh,
    scratch_types=[
        pltpu.VMEM((_CHUNK,), jnp.int32),
        pltpu.VMEM((_CHUNK,), jnp.int32),
        pltpu.VMEM((_CHUNK, _H), jnp.float32),
        pltpu.VMEM((_CHUNK, _H), jnp.float32),
        pltpu.SemaphoreType.DMA,
        pltpu.SemaphoreType.DMA,
    ],
)


_RB = 512
_GRID = _NPAD // _RB


def _dis_from(degp_ref):
    deg = degp_ref[0, :, 0:1] + degp_ref[1, :, 0:1] + 1.0
    return lax.rsqrt(deg)


def _tc1_body(x_ref, w1_ref, degp_ref, y1_ref):
    dis = _dis_from(degp_ref)
    y1_ref[...] = jnp.dot(x_ref[...], w1_ref[...],
                          preferred_element_type=jnp.float32) * dis


def _tc2_body(zp_ref, y1_ref, degp_ref, w2_ref, b1_ref, y2_ref):
    dis = _dis_from(degp_ref)
    h1 = jnp.maximum(
        (zp_ref[0] + zp_ref[1] + y1_ref[...]) * dis + b1_ref[...], 0.0)
    y2_ref[...] = jnp.dot(h1, w2_ref[...],
                          preferred_element_type=jnp.float32) * dis


def _tc3_body(zp_ref, y2_ref, degp_ref, b2_ref, tg_ref, bv_ref,
              wt_ref, bt_ref, wn_ref, bn_ref, we1t_ref, we1b_ref, be1_ref,
              nl_ref, p_ref, q_ref):
    dis = _dis_from(degp_ref)
    h2 = jnp.maximum(
        (zp_ref[0] + zp_ref[1] + y2_ref[...]) * dis + b2_ref[...], 0.0)
    half = _TDIM // 2
    k = lax.broadcasted_iota(jnp.int32, (1, half), 1).astype(jnp.float32)
    freqs = jnp.exp(k * (-math.log(10000.0) / half))
    args = tg_ref[...] * freqs
    emb = jnp.concatenate([jnp.sin(args), jnp.cos(args)], axis=-1)
    pre = jnp.dot(emb, wt_ref[...], preferred_element_type=jnp.float32) + bt_ref[...]
    temb = pre * (1.0 / (1.0 + jnp.exp(-pre)))
    bv = bv_ref[0, 0, :]
    onehot = (bv[:, None] == lax.broadcasted_iota(jnp.int32, (_RB, _B), 1)
              ).astype(jnp.float32)
    h = h2 + jnp.dot(onehot, temb, preferred_element_type=jnp.float32)
    nl_ref[...] = jnp.dot(h, wn_ref[...], preferred_element_type=jnp.float32) + bn_ref[...]
    p_ref[...] = jnp.dot(h, we1t_ref[...], preferred_element_type=jnp.float32) + be1_ref[...]
    q_ref[...] = jnp.dot(h, we1b_ref[...], preferred_element_type=jnp.float32)


_EB = 2048


def _tc4_body(r_ref, we2_ref, be2_ref, out_ref):
    r = r_ref[...]
    s = r * (1.0 / (1.0 + jnp.exp(-r)))
    out_ref[...] = jnp.dot(s, we2_ref[...],
                           preferred_element_type=jnp.float32) + be2_ref[...]


def _row_spec():
    return pl.BlockSpec((_RB, _H), lambda i: (i, 0))


def _full(shape):
    return pl.BlockSpec(shape, lambda i: tuple(0 for _ in shape))


_degp_spec = pl.BlockSpec((_NSC, _RB, _H), lambda i: (0, i, 0))
_zp_spec = pl.BlockSpec((_NSC, _RB, _H), lambda i: (0, i, 0))


def kernel(node_x_oh, edge_index, t_graph, batch_vec,
           W1, b1, W2, b2, Wt, bt, Wn, bn, We1, be1, We2, be2):
    f32 = jnp.float32
    x = jnp.pad(node_x_oh, ((0, _NPAD - _N), (0, 0)))
    pad = jnp.full((_EPAD - _E,), _DUMMY, jnp.int32)
    src_r = jnp.concatenate([edge_index[0], pad]).reshape(_NTILES, _NCH, _CHUNK)
    dst_r = jnp.concatenate([edge_index[1], pad]).reshape(_NTILES, _NCH, _CHUNK)
    batch_r = jnp.pad(batch_vec, (0, _NPAD - _N)).reshape(_GRID, 1, _RB)
    tg = t_graph.reshape(_B, 1)
    onesH = jnp.ones((_CHUNK, _H), f32)
    zerosH = jnp.zeros((_RPT, _H), f32)
    b1r = b1.reshape(1, _H)
    b2r = b2.reshape(1, _H)
    btr = bt.reshape(1, _H)
    bnr = bn.reshape(1, _H)
    be1r = be1.reshape(1, _H)
    be2r = be2.reshape(1, 16)
    We1t = We1[:_H]
    We1b = We1[_H:]

    degp = _deg_call(dst_r, onesH, zerosH)

    y1 = pl.pallas_call(
        _tc1_body,
        grid=(_GRID,),
        in_specs=[_row_spec(), _full((_H, _H)), _degp_spec],
        out_specs=_row_spec(),
        out_shape=jax.ShapeDtypeStruct((_NPAD, _H), f32),
    )(x, W1, degp)

    z1 = _spmm_call(y1, src_r, dst_r, zerosH)

    y2 = pl.pallas_call(
        _tc2_body,
        grid=(_GRID,),
        in_specs=[_zp_spec, _row_spec(), _degp_spec, _full((_H, _H)),
                  _full((1, _H))],
        out_specs=_row_spec(),
        out_shape=jax.ShapeDtypeStruct((_NPAD, _H), f32),
    )(z1, y1, degp, W2, b1r)

    z2 = _spmm_call(y2, src_r, dst_r, zerosH)

    nl, p, q = pl.pallas_call(
        _tc3_body,
        grid=(_GRID,),
        in_specs=[_zp_spec, _row_spec(), _degp_spec, _full((1, _H)),
                  _full((_B, 1)), pl.BlockSpec((1, 1, _RB), lambda i: (i, 0, 0)),
                  _full((_TDIM, _H)), _full((1, _H)),
                  _full((_H, _H)), _full((1, _H)),
                  _full((_H, _H)), _full((_H, _H)), _full((1, _H))],
        out_specs=[_row_spec(), _row_spec(), _row_spec()],
        out_shape=[jax.ShapeDtypeStruct((_NPAD, _H), f32)] * 3,
    )(z2, y2, degp, b2r, tg, batch_r, Wt, btr, Wn, bnr, We1t, We1b, be1r)

    r = _edge_call(p, q, src_r, dst_r)

    el = pl.pallas_call(
        _tc4_body,
        grid=(_EPAD // _EB,),
        in_specs=[pl.BlockSpec((_EB, _H), lambda i: (i, 0)),
                  _full((_H, 16)), _full((1, 16))],
        out_specs=pl.BlockSpec((_EB, 16), lambda i: (i, 0)),
        out_shape=jax.ShapeDtypeStruct((_EPAD, 16), f32),
    )(r, We2, be2r)

    return (nl[:_N], el[:_E])


# speedup vs baseline: 5.1934x; 1.2268x over previous
import math

import jax
import jax.numpy as jnp
from jax import lax
from jax.experimental import pallas as pl
from jax.experimental.pallas import tpu as pltpu
from jax.experimental.pallas import tpu_sc as plsc

_N = 10000
_E = 160000
_H = 128
_TDIM = 64
_B = 256

_NPAD = 10240
_DUMMY = _N
_NSC = 2
_NSUB = 16
_NTILES = _NSC * _NSUB
_CHUNK = 128
_NCH = 40
_EPAD = _NTILES * _NCH * _CHUNK
_RPT = _NPAD // _NSUB

_mesh = plsc.VectorSubcoreMesh(core_axis_name="c", subcore_axis_name="s")


def _deg_body(dst_hbm, ones_hbm, zeros_hbm, out_hbm, didx_v, ones_v, acc_sh,
              sem_s):
    cid = lax.axis_index("c")
    sid = lax.axis_index("s")
    wid = cid * _NSUB + sid
    pltpu.sync_copy(zeros_hbm, acc_sh.at[pl.ds(sid * _RPT, _RPT)])
    pltpu.sync_copy(ones_hbm, ones_v)
    pltpu.sync_copy(dst_hbm.at[wid], didx_v)
    plsc.subcore_barrier()

    def body(b, carry):
        cps = [pltpu.async_copy(ones_v, acc_sh.at[didx_v.at[b * 8 + t]],
                                sem_s, add=True) for t in range(8)]
        for cp in cps:
            cp.wait()
        return carry

    lax.fori_loop(0, _NCH // 8, body, 0)
    plsc.subcore_barrier()
    pltpu.sync_copy(acc_sh.at[pl.ds(sid * _RPT, _RPT)],
                    out_hbm.at[cid, pl.ds(sid * _RPT, _RPT)])


_deg_call = pl.kernel(
    _deg_body,
    out_type=jax.ShapeDtypeStruct((_NSC, _NPAD, _H), jnp.float32),
    mesh=_mesh,
    scratch_types=[
        pltpu.VMEM((_NCH, _CHUNK), jnp.int32),
        pltpu.VMEM((_CHUNK, _H), jnp.float32),
        pltpu.VMEM_SHARED((_NPAD, _H), jnp.float32),
        pltpu.SemaphoreType.DMA,
    ],
)


def _spmm_body(y_hbm, src_hbm, dst_hbm, zeros_hbm, out_hbm,
               sidx_v, didx_v, rows0, rows1, acc_sh, g0, g1, sem_s):
    cid = lax.axis_index("c")
    sid = lax.axis_index("s")
    wid = cid * _NSUB + sid
    rows = (rows0, rows1)
    gsems = (g0, g1)
    pltpu.sync_copy(zeros_hbm, acc_sh.at[pl.ds(sid * _RPT, _RPT)])
    pltpu.sync_copy(src_hbm.at[wid], sidx_v)
    pltpu.sync_copy(dst_hbm.at[wid], didx_v)
    plsc.subcore_barrier()
    for p in range(2):
        pltpu.async_copy(y_hbm.at[sidx_v.at[p]], rows[p], gsems[p])

    def body(k, carry):
        for p in range(2):
            j = k * 2 + p
            pltpu.make_async_copy(y_hbm.at[sidx_v.at[j]], rows[p],
                                  gsems[p]).wait()
            pltpu.async_copy(rows[p], acc_sh.at[didx_v.at[j]], sem_s,
                             add=True).wait()

            @pl.when(k < _NCH // 2 - 1)
            def _():
                pltpu.async_copy(y_hbm.at[sidx_v.at[j + 2]], rows[p], gsems[p])

        return carry

    lax.fori_loop(0, _NCH // 2, body, 0)
    plsc.subcore_barrier()
    pltpu.sync_copy(acc_sh.at[pl.ds(sid * _RPT, _RPT)],
                    out_hbm.at[cid, pl.ds(sid * _RPT, _RPT)])


_spmm_call = pl.kernel(
    _spmm_body,
    out_type=jax.ShapeDtypeStruct((_NSC, _NPAD, _H), jnp.float32),
    mesh=_mesh,
    scratch_types=[
        pltpu.VMEM((_NCH, _CHUNK), jnp.int32),
        pltpu.VMEM((_NCH, _CHUNK), jnp.int32),
        pltpu.VMEM((_CHUNK, _H), jnp.float32),
        pltpu.VMEM((_CHUNK, _H), jnp.float32),
        pltpu.VMEM_SHARED((_NPAD, _H), jnp.float32),
        pltpu.SemaphoreType.DMA,
        pltpu.SemaphoreType.DMA,
        pltpu.SemaphoreType.DMA,
    ],
)


def _edge_body(p_hbm, q_hbm, src_hbm, dst_hbm, out_hbm,
               sidx_v, didx_v, pb0, pb1, qb0, qb1, rb0, rb1,
               gp0, gp1, gq0, gq1, w0, w1):
    cid = lax.axis_index("c")
    sid = lax.axis_index("s")
    wid = cid * _NSUB + sid
    pbufs, qbufs, rbufs = (pb0, pb1), (qb0, qb1), (rb0, rb1)
    gpsems, gqsems, wsems = (gp0, gp1), (gq0, gq1), (w0, w1)
    pltpu.sync_copy(src_hbm.at[wid], sidx_v)
    pltpu.sync_copy(dst_hbm.at[wid], didx_v)
    for p in range(2):
        pltpu.async_copy(p_hbm.at[sidx_v.at[p]], pbufs[p], gpsems[p])
        pltpu.async_copy(q_hbm.at[didx_v.at[p]], qbufs[p], gqsems[p])

    def body(k, carry):
        for p in range(2):
            j = k * 2 + p
            pb, qb, rb = pbufs[p], qbufs[p], rbufs[p]
            pltpu.make_async_copy(p_hbm.at[sidx_v.at[j]], pb, gpsems[p]).wait()
            pltpu.make_async_copy(q_hbm.at[didx_v.at[j]], qb, gqsems[p]).wait()

            @pl.when(k > 0)
            def _():
                pltpu.make_async_copy(
                    rb, out_hbm.at[pl.ds(0, _CHUNK)], wsems[p]).wait()

            def rbody(r, c2):
                for c in range(_H // 16):
                    sl = pl.ds(c * 16, 16)
                    rb[r, sl] = pb[r, sl] + qb[r, sl]
                return c2

            lax.fori_loop(0, _CHUNK, rbody, 0)
            pltpu.async_copy(
                rb, out_hbm.at[pl.ds((wid * _NCH + j) * _CHUNK, _CHUNK)],
                wsems[p])

            @pl.when(k < _NCH // 2 - 1)
            def _():
                pltpu.async_copy(p_hbm.at[sidx_v.at[j + 2]], pb, gpsems[p])
                pltpu.async_copy(q_hbm.at[didx_v.at[j + 2]], qb, gqsems[p])

        return carry

    lax.fori_loop(0, _NCH // 2, body, 0)
    for p in range(2):
        pltpu.make_async_copy(rbufs[p], out_hbm.at[pl.ds(0, _CHUNK)],
                              wsems[p]).wait()


_edge_call = pl.kernel(
    _edge_body,
    out_type=jax.ShapeDtypeStruct((_EPAD, _H), jnp.float32),
    mesh=_mesh,
    scratch_types=[
        pltpu.VMEM((_NCH, _CHUNK), jnp.int32),
        pltpu.VMEM((_NCH, _CHUNK), jnp.int32),
        pltpu.VMEM((_CHUNK, _H), jnp.float32),
        pltpu.VMEM((_CHUNK, _H), jnp.float32),
        pltpu.VMEM((_CHUNK, _H), jnp.float32),
        pltpu.VMEM((_CHUNK, _H), jnp.float32),
        pltpu.VMEM((_CHUNK, _H), jnp.float32),
        pltpu.VMEM((_CHUNK, _H), jnp.float32),
        pltpu.SemaphoreType.DMA,
        pltpu.SemaphoreType.DMA,
        pltpu.SemaphoreType.DMA,
        pltpu.SemaphoreType.DMA,
        pltpu.SemaphoreType.DMA,
        pltpu.SemaphoreType.DMA,
    ],
)


_RB = 512
_GRID = _NPAD // _RB


def _dis_from(degp_ref):
    deg = degp_ref[0, :, 0:1] + degp_ref[1, :, 0:1] + 1.0
    return lax.rsqrt(deg)


def _tc1_body(x_ref, w1_ref, degp_ref, y1_ref):
    dis = _dis_from(degp_ref)
    y1_ref[...] = jnp.dot(x_ref[...], w1_ref[...],
                          preferred_element_type=jnp.float32) * dis


def _tc2_body(zp_ref, y1_ref, degp_ref, w2_ref, b1_ref, y2_ref):
    dis = _dis_from(degp_ref)
    h1 = jnp.maximum(
        (zp_ref[0] + zp_ref[1] + y1_ref[...]) * dis + b1_ref[...], 0.0)
    y2_ref[...] = jnp.dot(h1, w2_ref[...],
                          preferred_element_type=jnp.float32) * dis


def _tc3_body(zp_ref, y2_ref, degp_ref, b2_ref, tg_ref, bv_ref,
              wt_ref, bt_ref, wn_ref, bn_ref, we1t_ref, we1b_ref, be1_ref,
              nl_ref, p_ref, q_ref):
    dis = _dis_from(degp_ref)
    h2 = jnp.maximum(
        (zp_ref[0] + zp_ref[1] + y2_ref[...]) * dis + b2_ref[...], 0.0)
    half = _TDIM // 2
    k = lax.broadcasted_iota(jnp.int32, (1, half), 1).astype(jnp.float32)
    freqs = jnp.exp(k * (-math.log(10000.0) / half))
    args = tg_ref[...] * freqs
    emb = jnp.concatenate([jnp.sin(args), jnp.cos(args)], axis=-1)
    pre = jnp.dot(emb, wt_ref[...], preferred_element_type=jnp.float32) + bt_ref[...]
    temb = pre * (1.0 / (1.0 + jnp.exp(-pre)))
    bv = bv_ref[0, 0, :]
    onehot = (bv[:, None] == lax.broadcasted_iota(jnp.int32, (_RB, _B), 1)
              ).astype(jnp.float32)
    h = h2 + jnp.dot(onehot, temb, preferred_element_type=jnp.float32)
    nl_ref[...] = jnp.dot(h, wn_ref[...], preferred_element_type=jnp.float32) + bn_ref[...]
    p_ref[...] = jnp.dot(h, we1t_ref[...], preferred_element_type=jnp.float32) + be1_ref[...]
    q_ref[...] = jnp.dot(h, we1b_ref[...], preferred_element_type=jnp.float32)


_EB = 2048


def _tc4_body(r_ref, we2_ref, be2_ref, out_ref):
    r = r_ref[...]
    s = r * (1.0 / (1.0 + jnp.exp(-r)))
    out_ref[...] = jnp.dot(s, we2_ref[...],
                           preferred_element_type=jnp.float32) + be2_ref[...]


def _row_spec():
    return pl.BlockSpec((_RB, _H), lambda i: (i, 0))


def _full(shape):
    return pl.BlockSpec(shape, lambda i: tuple(0 for _ in shape))


_degp_spec = pl.BlockSpec((_NSC, _RB, _H), lambda i: (0, i, 0))
_zp_spec = pl.BlockSpec((_NSC, _RB, _H), lambda i: (0, i, 0))


def kernel(node_x_oh, edge_index, t_graph, batch_vec,
           W1, b1, W2, b2, Wt, bt, Wn, bn, We1, be1, We2, be2):
    f32 = jnp.float32
    x = jnp.pad(node_x_oh, ((0, _NPAD - _N), (0, 0)))
    pad = jnp.full((_EPAD - _E,), _DUMMY, jnp.int32)
    src_r = jnp.concatenate([edge_index[0], pad]).reshape(_NTILES, _NCH, _CHUNK)
    dst_r = jnp.concatenate([edge_index[1], pad]).reshape(_NTILES, _NCH, _CHUNK)
    batch_r = jnp.pad(batch_vec, (0, _NPAD - _N)).reshape(_GRID, 1, _RB)
    tg = t_graph.reshape(_B, 1)
    onesH = jnp.ones((_CHUNK, _H), f32)
    zerosH = jnp.zeros((_RPT, _H), f32)
    b1r = b1.reshape(1, _H)
    b2r = b2.reshape(1, _H)
    btr = bt.reshape(1, _H)
    bnr = bn.reshape(1, _H)
    be1r = be1.reshape(1, _H)
    be2r = be2.reshape(1, 16)
    We1t = We1[:_H]
    We1b = We1[_H:]

    degp = _deg_call(dst_r, onesH, zerosH)

    y1 = pl.pallas_call(
        _tc1_body,
        grid=(_GRID,),
        in_specs=[_row_spec(), _full((_H, _H)), _degp_spec],
        out_specs=_row_spec(),
        out_shape=jax.ShapeDtypeStruct((_NPAD, _H), f32),
    )(x, W1, degp)

    z1 = _spmm_call(y1, src_r, dst_r, zerosH)

    y2 = pl.pallas_call(
        _tc2_body,
        grid=(_GRID,),
        in_specs=[_zp_spec, _row_spec(), _degp_spec, _full((_H, _H)),
                  _full((1, _H))],
        out_specs=_row_spec(),
        out_shape=jax.ShapeDtypeStruct((_NPAD, _H), f32),
    )(z1, y1, degp, W2, b1r)

    z2 = _spmm_call(y2, src_r, dst_r, zerosH)

    nl, p, q = pl.pallas_call(
        _tc3_body,
        grid=(_GRID,),
        in_specs=[_zp_spec, _row_spec(), _degp_spec, _full((1, _H)),
                  _full((_B, 1)), pl.BlockSpec((1, 1, _RB), lambda i: (i, 0, 0)),
                  _full((_TDIM, _H)), _full((1, _H)),
                  _full((_H, _H)), _full((1, _H)),
                  _full((_H, _H)), _full((_H, _H)), _full((1, _H))],
        out_specs=[_row_spec(), _row_spec(), _row_spec()],
        out_shape=[jax.ShapeDtypeStruct((_NPAD, _H), f32)] * 3,
    )(z2, y2, degp, b2r, tg, batch_r, Wt, btr, Wn, bnr, We1t, We1b, be1r)

    r = _edge_call(p, q, src_r, dst_r)

    el = pl.pallas_call(
        _tc4_body,
        grid=(_EPAD // _EB,),
        in_specs=[pl.BlockSpec((_EB, _H), lambda i: (i, 0)),
                  _full((_H, 16)), _full((1, 16))],
        out_specs=pl.BlockSpec((_EB, 16), lambda i: (i, 0)),
        out_shape=jax.ShapeDtypeStruct((_EPAD, 16), f32),
    )(r, We2, be2r)

    return (nl[:_N], el[:_E])
